# Initial kernel scaffold; baseline (speedup 1.0000x reference)
import jax
import jax.numpy as jnp
from jax.experimental import pallas as pl


def kernel(x, table):
    raise NotImplementedError("write your pallas kernel here")


# speedup vs baseline: 2.0145x; 2.0145x over previous
import functools

import jax
import jax.numpy as jnp
from jax import lax
from jax.experimental import pallas as pl
from jax.experimental.pallas import tpu as pltpu
from jax.experimental.pallas import tpu_sc as plsc

_BATCH = 16384
_SEQ = 200
_DIM = 64
_LP = 208
_NCH = _LP // 16
_NW = 32
_ROWS_PW = _BATCH // _NW
_R = 8
_STEPS = _ROWS_PW // _R
_IDXN = _R * _LP
_NSTREAM = _IDXN // 128


def _sc_body(x_hbm, table_hbm, out_hbm, xbuf, idx, gbuf, sem):
    c = lax.axis_index("c")
    s = lax.axis_index("s")
    wid = s * 2 + c
    base0 = wid * _ROWS_PW

    zeros16 = jnp.zeros((16,), jnp.int32)
    for r in range(_R):
        xbuf[pl.ds(r * _LP + 192, 16)] = zeros16

    def step(it, carry_none):
        base = base0 + it * _R
        for r in range(_R):
            pltpu.sync_copy(x_hbm.at[base + r],
                            xbuf.at[pl.ds(r * _LP, _SEQ)])
        for r in range(_R):
            carry = jnp.int32(0)
            for i in range(_NCH):
                v = xbuf[pl.ds(r * _LP + i * 16, 16)]
                m = jnp.minimum(jnp.abs(v), 1)
                cs = jnp.cumsum(m)
                idx[pl.ds(r * _LP + i * 16, 16)] = (carry + cs) * m
                carry = carry + jnp.sum(m)
        copies = [
            pltpu.make_async_copy(
                table_hbm.at[idx.at[pl.ds(j * 128, 128)]],
                gbuf.at[pl.ds(j * 128, 128)],
                sem,
            )
            for j in range(_NSTREAM)
        ]
        for cp in copies:
            cp.start()
        for cp in copies:
            cp.wait()
        for r in range(_R):
            pltpu.sync_copy(gbuf.at[pl.ds(r * _LP, _SEQ)],
                            out_hbm.at[base + r])
        return carry_none

    lax.fori_loop(0, _STEPS, step, None)


def kernel(x, table):
    mesh = plsc.VectorSubcoreMesh(core_axis_name="c", subcore_axis_name="s")
    f = functools.partial(
        pl.kernel,
        mesh=mesh,
        compiler_params=pltpu.CompilerParams(use_tc_tiling_on_sc=False,
                                             needs_layout_passes=False),
        out_type=jax.ShapeDtypeStruct((_BATCH, _SEQ, _DIM), jnp.float32),
        scratch_types=[
            pltpu.VMEM((_R * _LP,), jnp.int32),
            pltpu.VMEM((_IDXN,), jnp.int32),
            pltpu.VMEM((_IDXN, _DIM), jnp.float32),
            pltpu.SemaphoreType.DMA,
        ],
    )(_sc_body)
    return f(x, table)


# speedup vs baseline: 5.7588x; 2.8587x over previous
import functools

import jax
import jax.numpy as jnp
from jax import lax
from jax.experimental import pallas as pl
from jax.experimental.pallas import tpu as pltpu
from jax.experimental.pallas import tpu_sc as plsc

_BATCH = 16384
_SEQ = 200
_DIM = 64
_TROWS = 201
_LP = 208
_NCH = _LP // 16
_NW = 32
_ROWS_PW = _BATCH // _NW
_R = 4
_STEPS = _ROWS_PW // _R
_IDXN = _R * _LP
_GCHUNK = 104
_NSTREAM = _IDXN // _GCHUNK


def _x_copies(x_hbm, xb, base):
    return [
        pltpu.make_async_copy(x_hbm.at[base + r],
                              xb[0].at[pl.ds(r * _LP, _SEQ)],
                              xb[1])
        for r in range(_R)
    ]


def _g_copies(tbuf, idx, gb):
    return [
        pltpu.make_async_copy(
            tbuf.at[idx[0].at[pl.ds(j * _GCHUNK, _GCHUNK)]],
            gb[0].at[pl.ds(j * _GCHUNK, _GCHUNK)],
            gb[1],
        )
        for j in range(_NSTREAM)
    ]


def _o_copies(out_hbm, gb, base):
    return [
        pltpu.make_async_copy(gb[0].at[pl.ds(r * _LP, _SEQ)],
                              out_hbm.at[base + r],
                              gb[2])
        for r in range(_R)
    ]


def _sc_body(x_hbm, table_hbm, out_hbm,
             tbuf, xb0, xb1, id0, id1, gb0, gb1,
             semt, sx0, sx1, sg0, sg1, so0, so1):
    c = lax.axis_index("c")
    s = lax.axis_index("s")
    wid = s * 2 + c
    base0 = wid * _ROWS_PW

    xbs = [(xb0, sx0), (xb1, sx1)]
    ids = [(id0, None), (id1, None)]
    gbs = [(gb0, sg0, so0), (gb1, sg1, so1)]

    @pl.when(s == 0)
    def _():
        pltpu.make_async_copy(table_hbm, tbuf, semt).start()

    zeros16 = jnp.zeros((16,), jnp.int32)
    for xb, _ in xbs:
        for r in range(_R):
            xb[pl.ds(r * _LP + 192, 16)] = zeros16

    @pl.when(s == 0)
    def _():
        pltpu.make_async_copy(table_hbm, tbuf, semt).wait()
    plsc.subcore_barrier()

    for cp in _x_copies(x_hbm, xbs[0], base0):
        cp.start()

    def substep(it, p):
        xb, sx = xbs[p]
        idx = ids[p][0]
        gb, sg, so = gbs[p]
        base = base0 + it * _R
        for cp in _x_copies(x_hbm, (xb, sx), base):
            cp.wait()
        for r in range(_R):
            carry = jnp.int32(0)
            for i in range(_NCH):
                v = xb[pl.ds(r * _LP + i * 16, 16)]
                m = jnp.minimum(jnp.abs(v), 1)
                cs = jnp.cumsum(m)
                idx[pl.ds(r * _LP + i * 16, 16)] = (carry + cs) * m
                carry = carry + cs[15]
        @pl.when(it >= 2)
        def _():
            for cp in _o_copies(out_hbm, (gb, sg, so),
                                base0 + (it - 2) * _R):
                cp.wait()
        gcps = _g_copies(tbuf, (idx, None), (gb, sg))
        for cp in gcps:
            cp.start()
        @pl.when(it + 1 < _STEPS)
        def _():
            for cp in _x_copies(x_hbm, xbs[1 - p], base + _R):
                cp.start()
        for cp in gcps:
            cp.wait()
        for cp in _o_copies(out_hbm, (gb, sg, so), base):
            cp.start()

    def step2(i2, carry_none):
        substep(i2 * 2, 0)
        substep(i2 * 2 + 1, 1)
        return carry_none

    lax.fori_loop(0, _STEPS // 2, step2, None)

    for p, it in ((0, _STEPS - 2), (1, _STEPS - 1)):
        gb, sg, so = gbs[p]
        for cp in _o_copies(out_hbm, (gb, sg, so), base0 + it * _R):
            cp.wait()


def kernel(x, table):
    mesh = plsc.VectorSubcoreMesh(core_axis_name="c", subcore_axis_name="s")
    f = functools.partial(
        pl.kernel,
        mesh=mesh,
        compiler_params=pltpu.CompilerParams(use_tc_tiling_on_sc=False,
                                             needs_layout_passes=False),
        out_type=jax.ShapeDtypeStruct((_BATCH, _SEQ, _DIM), jnp.float32),
        scratch_types=[
            pltpu.VMEM_SHARED((_TROWS, _DIM), jnp.float32),
            pltpu.VMEM((_IDXN,), jnp.int32),
            pltpu.VMEM((_IDXN,), jnp.int32),
            pltpu.VMEM((_IDXN,), jnp.int32),
            pltpu.VMEM((_IDXN,), jnp.int32),
            pltpu.VMEM((_IDXN, _DIM), jnp.float32),
            pltpu.VMEM((_IDXN, _DIM), jnp.float32),
            pltpu.SemaphoreType.DMA,
            pltpu.SemaphoreType.DMA,
            pltpu.SemaphoreType.DMA,
            pltpu.SemaphoreType.DMA,
            pltpu.SemaphoreType.DMA,
            pltpu.SemaphoreType.DMA,
            pltpu.SemaphoreType.DMA,
        ],
    )(_sc_body)
    return f(x, table)


# speedup vs baseline: 5.7672x; 1.0015x over previous
import functools

import jax
import jax.numpy as jnp
from jax import lax
from jax.experimental import pallas as pl
from jax.experimental.pallas import tpu as pltpu
from jax.experimental.pallas import tpu_sc as plsc

_BATCH = 16384
_SEQ = 200
_DIM = 64
_TROWS = 201
_LP = 208
_NCH = _LP // 16
_NW = 32
_ROWS_PW = _BATCH // _NW
_R = 4
_STEPS = _ROWS_PW // _R
_IDXN = _R * _LP
_GCHUNK = 104


def _x_copy(x_hbm, xb, sx, base):
    return pltpu.make_async_copy(x_hbm.at[pl.ds(base, _R)],
                                 xb.at[pl.ds(0, _R), pl.ds(0, _SEQ)], sx)


def _g_copies(tbuf, idx, gb, sg):
    return [
        pltpu.make_async_copy(
            tbuf.at[idx.at[pl.ds((2 * r + h) * _GCHUNK, _GCHUNK)]],
            gb.at[r, pl.ds(h * _GCHUNK, _GCHUNK)],
            sg,
        )
        for r in range(_R)
        for h in range(2)
    ]


def _o_copy(out_hbm, gb, so, base):
    return pltpu.make_async_copy(gb.at[pl.ds(0, _R), pl.ds(0, _SEQ)],
                                 out_hbm.at[pl.ds(base, _R)], so)


def _sc_body(x_hbm, table_hbm, out_hbm,
             tbuf, xb0, xb1, id0, id1, gb0, gb1,
             semt, sx0, sx1, sg0, sg1, so0, so1):
    c = lax.axis_index("c")
    s = lax.axis_index("s")
    wid = s * 2 + c
    base0 = wid * _ROWS_PW

    xbs = [(xb0, sx0), (xb1, sx1)]
    ids = [id0, id1]
    gbs = [(gb0, sg0, so0), (gb1, sg1, so1)]

    @pl.when(s == 0)
    def _():
        pltpu.make_async_copy(table_hbm, tbuf, semt).start()

    zeros16 = jnp.zeros((16,), jnp.int32)
    for xb, _ in xbs:
        for r in range(_R):
            xb[r, pl.ds(192, 16)] = zeros16

    @pl.when(s == 0)
    def _():
        pltpu.make_async_copy(table_hbm, tbuf, semt).wait()
    plsc.subcore_barrier()

    _x_copy(x_hbm, xb0, sx0, base0).start()

    def substep(it, p):
        q = 1 - p
        xb, sx = xbs[p]
        idx = ids[p]
        gb, sg, so = gbs[p]
        gbq, sgq, soq = gbs[q]
        base = base0 + it * _R
        _x_copy(x_hbm, xb, sx, base).wait()
        for r in range(_R):
            carry = jnp.int32(0)
            for i in range(_NCH):
                v = xb[r, pl.ds(i * 16, 16)]
                m = jnp.minimum(jnp.abs(v), 1)
                cs = jnp.cumsum(m)
                idx[pl.ds(r * _LP + i * 16, 16)] = (carry + cs) * m
                carry = carry + cs[15]
        @pl.when(it >= 2)
        def _():
            _o_copy(out_hbm, gb, so, base0 + (it - 2) * _R).wait()
        for cp in _g_copies(tbuf, idx, gb, sg):
            cp.start()
        @pl.when(it + 1 < _STEPS)
        def _():
            _x_copy(x_hbm, xbs[q][0], xbs[q][1], base + _R).start()
        @pl.when(it >= 1)
        def _():
            for cp in _g_copies(tbuf, ids[q], gbq, sgq):
                cp.wait()
            _o_copy(out_hbm, gbq, soq, base0 + (it - 1) * _R).start()

    def step2(i2, carry_none):
        substep(i2 * 2, 0)
        substep(i2 * 2 + 1, 1)
        return carry_none

    lax.fori_loop(0, _STEPS // 2, step2, None)

    last = _STEPS - 1
    gb, sg, so = gbs[1]
    for cp in _g_copies(tbuf, ids[1], gb, sg):
        cp.wait()
    _o_copy(out_hbm, gb, so, base0 + last * _R).start()
    _o_copy(out_hbm, gbs[0][0], gbs[0][2], base0 + (last - 1) * _R).wait()
    _o_copy(out_hbm, gb, so, base0 + last * _R).wait()


def kernel(x, table):
    mesh = plsc.VectorSubcoreMesh(core_axis_name="c", subcore_axis_name="s")
    f = functools.partial(
        pl.kernel,
        mesh=mesh,
        compiler_params=pltpu.CompilerParams(use_tc_tiling_on_sc=False,
                                             needs_layout_passes=False),
        out_type=jax.ShapeDtypeStruct((_BATCH, _SEQ, _DIM), jnp.float32),
        scratch_types=[
            pltpu.VMEM_SHARED((_TROWS, _DIM), jnp.float32),
            pltpu.VMEM((_R, _LP), jnp.int32),
            pltpu.VMEM((_R, _LP), jnp.int32),
            pltpu.VMEM((_IDXN,), jnp.int32),
            pltpu.VMEM((_IDXN,), jnp.int32),
            pltpu.VMEM((_R, _LP, _DIM), jnp.float32),
            pltpu.VMEM((_R, _LP, _DIM), jnp.float32),
            pltpu.SemaphoreType.DMA,
            pltpu.SemaphoreType.DMA,
            pltpu.SemaphoreType.DMA,
            pltpu.SemaphoreType.DMA,
            pltpu.SemaphoreType.DMA,
            pltpu.SemaphoreType.DMA,
            pltpu.SemaphoreType.DMA,
        ],
    )(_sc_body)
    return f(x, table)
